# Initial kernel scaffold; baseline (speedup 1.0000x reference)
import functools

import jax
import jax.numpy as jnp
from jax import lax
from jax.experimental import pallas as pl
from jax.experimental.pallas import tpu as pltpu
from jax.experimental.pallas import tpu_sc as plsc

_NV, _NE, _E, _D = 10000, 5000, 320000, 128
_NC, _NS, _L = 2, 16, 16
_NW = _NC * _NS
_B = 80


def _dense_in_body(v_ref, w_ref, b_ref, nw_ref, x_ref):
    x = lax.dot_general(v_ref[...], w_ref[...],
                        (((1,), (1,)), ((), ())),
                        preferred_element_type=jnp.float32)
    x_ref[...] = jnp.maximum(x + b_ref[...], 0.0) * nw_ref[...]


def _dense_in(v, W, b2, nw, blk):
    n = v.shape[0]
    grid = n // blk
    return pl.pallas_call(
        _dense_in_body,
        grid=(grid,),
        in_specs=[
            pl.BlockSpec((blk, _D), lambda i: (i, 0)),
            pl.BlockSpec((_D, _D), lambda i: (0, 0)),
            pl.BlockSpec((1, _D), lambda i: (0, 0)),
            pl.BlockSpec((blk, 1), lambda i: (i, 0)),
        ],
        out_specs=pl.BlockSpec((blk, _D), lambda i: (i, 0)),
        out_shape=jax.ShapeDtypeStruct((n, _D), jnp.float32),
    )(v, W, b2, nw)


def _dense_mid_body(e_ref, s0_ref, s1_ref, ers_ref, w_ref, b_ref, ew_ref,
                    e1_ref, y_ref):
    e1 = (e_ref[...] + s0_ref[...] + s1_ref[...]) / ers_ref[...]
    e1_ref[...] = e1
    y = lax.dot_general(e1, w_ref[...], (((1,), (1,)), ((), ())),
                        preferred_element_type=jnp.float32)
    y_ref[...] = jnp.maximum(y + b_ref[...], 0.0) * ew_ref[...]


def _dense_mid(e, s0, s1, ers, W, b2, ew, blk):
    n = e.shape[0]
    grid = n // blk
    row = lambda i: (i, 0)
    fixed = lambda i: (0, 0)
    return pl.pallas_call(
        _dense_mid_body,
        grid=(grid,),
        in_specs=[
            pl.BlockSpec((blk, _D), row),
            pl.BlockSpec((blk, _D), row),
            pl.BlockSpec((blk, _D), row),
            pl.BlockSpec((blk, 1), row),
            pl.BlockSpec((_D, _D), fixed),
            pl.BlockSpec((1, _D), fixed),
            pl.BlockSpec((blk, 1), row),
        ],
        out_specs=[pl.BlockSpec((blk, _D), row), pl.BlockSpec((blk, _D), row)],
        out_shape=[jax.ShapeDtypeStruct((n, _D), jnp.float32),
                   jax.ShapeDtypeStruct((n, _D), jnp.float32)],
    )(e, s0, s1, ers, W, b2, ew)


def _combine_body(v_ref, nw_ref, t0_ref, t1_ref, nrs_ref, out_ref):
    out_ref[...] = (v_ref[...] * nw_ref[...] + t0_ref[...] + t1_ref[...]) \
        / nrs_ref[...]


def _combine(v, nw, t0, t1, nrs, blk):
    n = v.shape[0]
    grid = n // blk
    row = lambda i: (i, 0)
    return pl.pallas_call(
        _combine_body,
        grid=(grid,),
        in_specs=[
            pl.BlockSpec((blk, _D), row),
            pl.BlockSpec((blk, 1), row),
            pl.BlockSpec((blk, _D), row),
            pl.BlockSpec((blk, _D), row),
            pl.BlockSpec((blk, 1), row),
        ],
        out_specs=pl.BlockSpec((blk, _D), row),
        out_shape=jax.ShapeDtypeStruct((n, _D), jnp.float32),
    )(v, nw, t0, t1, nrs)


def _make_scatter(n_dst_pad):
    per_w = _E // _NW
    n_chunks = per_w // _B
    rows_per_sub = n_dst_pad // _NS
    n_zero = (rows_per_sub + _B - 1) // _B
    mesh = plsc.VectorSubcoreMesh(core_axis_name="c", subcore_axis_name="s")

    @functools.partial(
        pl.kernel, mesh=mesh,
        out_type=jax.ShapeDtypeStruct((_NC, n_dst_pad, _D), jnp.float32),
        scratch_types=[
            pltpu.VMEM((_B,), jnp.int32),
            pltpu.VMEM((_B,), jnp.int32),
            pltpu.VMEM((_B,), jnp.float32),
            pltpu.VMEM((_B, _D), jnp.float32),
            pltpu.VMEM_SHARED((n_dst_pad, _D), jnp.float32),
            pltpu.SemaphoreType.DMA,
        ],
    )
    def k(x_hbm, si_hbm, w_hbm, di_hbm, out_hbm,
          si_v, di_v, w_v, rows_v, acc_sh, sem):
        c = lax.axis_index("c")
        s = lax.axis_index("s")
        wid = s * _NC + c
        zero16 = jnp.zeros((_L,), jnp.float32)

        def zero_rows(r, carry):
            for j in range(_D // _L):
                rows_v[r, pl.ds(j * _L, _L)] = zero16
            return carry
        lax.fori_loop(0, _B, zero_rows, 0)
        for t in range(n_zero):
            base = s * rows_per_sub + t * _B
            nrows = min(_B, rows_per_sub - t * _B)
            pltpu.sync_copy(rows_v.at[pl.ds(0, nrows)],
                            acc_sh.at[pl.ds(base, nrows)])
        plsc.subcore_barrier()

        def chunk(t, carry):
            base = wid * per_w + t * _B
            pltpu.sync_copy(si_hbm.at[pl.ds(base, _B)], si_v)
            pltpu.sync_copy(di_hbm.at[pl.ds(base, _B)], di_v)
            pltpu.sync_copy(w_hbm.at[pl.ds(base, _B)], w_v)
            pltpu.async_copy(x_hbm.at[si_v], rows_v, sem).wait()

            def scale(r, carry2):
                wv = plsc.load_gather(
                    w_v, [jnp.full((_L,), 0, jnp.int32) + r])
                for j in range(_D // _L):
                    sl = pl.ds(j * _L, _L)
                    rows_v[r, sl] = rows_v[r, sl] * wv
                return carry2
            lax.fori_loop(0, _B, scale, 0)
            pltpu.sync_copy(rows_v, acc_sh.at[di_v], add=True)
            return carry
        lax.fori_loop(0, n_chunks, chunk, 0)
        plsc.subcore_barrier()
        pltpu.sync_copy(acc_sh.at[pl.ds(s * rows_per_sub, rows_per_sub)],
                        out_hbm.at[c, pl.ds(s * rows_per_sub, rows_per_sub)])

    return k


_scatter_e = _make_scatter(5120)
_scatter_v = _make_scatter(10240)


def kernel(v, e, W_v2e, b_v2e, W_e2v, b_e2v, n_weight, e_weight,
           n_reg_weight, e_reg_weight, e_reg_sum, n_reg_sum,
           vidx, eidx, ve_lists):
    ve0 = ve_lists[:, 0]
    ve1 = ve_lists[:, 1]
    nrw = n_reg_weight[:, 0]
    erw = e_reg_weight[:, 0]

    x = _dense_in(v, W_v2e, b_v2e.reshape(1, _D), n_weight, 2000)
    s = _scatter_e(x, ve0, nrw, eidx)
    e1, y = _dense_mid(e, s[0, :_NE], s[1, :_NE], e_reg_sum,
                       W_e2v, b_e2v.reshape(1, _D), e_weight, 1000)
    t = _scatter_v(y, ve1, erw, vidx)
    v2 = _combine(v, n_weight, t[0, :_NV], t[1, :_NV], n_reg_sum, 2000)
    return (v2, e1)


# speedup vs baseline: 3.6789x; 3.6789x over previous
import functools

import jax
import jax.numpy as jnp
from jax import lax
from jax.experimental import pallas as pl
from jax.experimental.pallas import tpu as pltpu
from jax.experimental.pallas import tpu_sc as plsc

_NV, _NE, _E, _D = 10000, 5000, 320000, 128
_NC, _NS, _L = 2, 16, 16
_NW = _NC * _NS
_B = 80


def _dense_in_body(v_ref, w_ref, b_ref, nw_ref, x_ref):
    x = lax.dot_general(v_ref[...], w_ref[...],
                        (((1,), (1,)), ((), ())),
                        preferred_element_type=jnp.float32)
    x_ref[...] = jnp.maximum(x + b_ref[...], 0.0) * nw_ref[...]


def _dense_in(v, W, b2, nw, blk):
    n = v.shape[0]
    grid = n // blk
    return pl.pallas_call(
        _dense_in_body,
        grid=(grid,),
        in_specs=[
            pl.BlockSpec((blk, _D), lambda i: (i, 0)),
            pl.BlockSpec((_D, _D), lambda i: (0, 0)),
            pl.BlockSpec((1, _D), lambda i: (0, 0)),
            pl.BlockSpec((blk, 1), lambda i: (i, 0)),
        ],
        out_specs=pl.BlockSpec((blk, _D), lambda i: (i, 0)),
        out_shape=jax.ShapeDtypeStruct((n, _D), jnp.float32),
    )(v, W, b2, nw)


def _dense_mid_body(e_ref, s0_ref, s1_ref, ers_ref, w_ref, b_ref, ew_ref,
                    e1_ref, y_ref):
    e1 = (e_ref[...] + s0_ref[...] + s1_ref[...]) / ers_ref[...]
    e1_ref[...] = e1
    y = lax.dot_general(e1, w_ref[...], (((1,), (1,)), ((), ())),
                        preferred_element_type=jnp.float32)
    y_ref[...] = jnp.maximum(y + b_ref[...], 0.0) * ew_ref[...]


def _dense_mid(e, s0, s1, ers, W, b2, ew, blk):
    n = e.shape[0]
    grid = n // blk
    row = lambda i: (i, 0)
    fixed = lambda i: (0, 0)
    return pl.pallas_call(
        _dense_mid_body,
        grid=(grid,),
        in_specs=[
            pl.BlockSpec((blk, _D), row),
            pl.BlockSpec((blk, _D), row),
            pl.BlockSpec((blk, _D), row),
            pl.BlockSpec((blk, 1), row),
            pl.BlockSpec((_D, _D), fixed),
            pl.BlockSpec((1, _D), fixed),
            pl.BlockSpec((blk, 1), row),
        ],
        out_specs=[pl.BlockSpec((blk, _D), row), pl.BlockSpec((blk, _D), row)],
        out_shape=[jax.ShapeDtypeStruct((n, _D), jnp.float32),
                   jax.ShapeDtypeStruct((n, _D), jnp.float32)],
    )(e, s0, s1, ers, W, b2, ew)


def _combine_body(v_ref, nw_ref, t0_ref, t1_ref, nrs_ref, out_ref):
    out_ref[...] = (v_ref[...] * nw_ref[...] + t0_ref[...] + t1_ref[...]) \
        / nrs_ref[...]


def _combine(v, nw, t0, t1, nrs, blk):
    n = v.shape[0]
    grid = n // blk
    row = lambda i: (i, 0)
    return pl.pallas_call(
        _combine_body,
        grid=(grid,),
        in_specs=[
            pl.BlockSpec((blk, _D), row),
            pl.BlockSpec((blk, 1), row),
            pl.BlockSpec((blk, _D), row),
            pl.BlockSpec((blk, _D), row),
            pl.BlockSpec((blk, 1), row),
        ],
        out_specs=pl.BlockSpec((blk, _D), row),
        out_shape=jax.ShapeDtypeStruct((n, _D), jnp.float32),
    )(v, nw, t0, t1, nrs)


def _make_scatter(n_dst_pad):
    per_w = _E // _NW
    n_chunks = per_w // _B
    rows_per_sub = n_dst_pad // _NS
    n_zero = (rows_per_sub + _B - 1) // _B
    mesh = plsc.VectorSubcoreMesh(core_axis_name="c", subcore_axis_name="s")

    @functools.partial(
        pl.kernel, mesh=mesh,
        out_type=jax.ShapeDtypeStruct((_NC, n_dst_pad, _D), jnp.float32),
        scratch_types=[
            pltpu.VMEM((_B,), jnp.int32),
            pltpu.VMEM((_B,), jnp.int32),
            pltpu.VMEM((_B,), jnp.float32),
            pltpu.VMEM((_B, _D), jnp.float32),
            pltpu.VMEM_SHARED((n_dst_pad, _D), jnp.float32),
            pltpu.SemaphoreType.DMA,
        ],
    )
    def k(x_hbm, si_hbm, w_hbm, di_hbm, out_hbm,
          si_v, di_v, w_v, rows_v, acc_sh, sem):
        c = lax.axis_index("c")
        s = lax.axis_index("s")
        wid = s * _NC + c
        zero16 = jnp.zeros((_L,), jnp.float32)

        def zero_rows(r, carry):
            for j in range(_D // _L):
                rows_v[r, pl.ds(j * _L, _L)] = zero16
            return carry
        lax.fori_loop(0, _B, zero_rows, 0)
        for t in range(n_zero):
            base = s * rows_per_sub + t * _B
            nrows = min(_B, rows_per_sub - t * _B)
            pltpu.sync_copy(rows_v.at[pl.ds(0, nrows)],
                            acc_sh.at[pl.ds(base, nrows)])
        plsc.subcore_barrier()

        def chunk(t, carry):
            base = wid * per_w + t * _B
            pltpu.sync_copy(si_hbm.at[pl.ds(base, _B)], si_v)
            pltpu.sync_copy(di_hbm.at[pl.ds(base, _B)], di_v)
            pltpu.sync_copy(w_hbm.at[pl.ds(base, _B)], w_v)
            pltpu.async_copy(x_hbm.at[si_v], rows_v, sem).wait()

            def scale(g, carry2):
                wg = w_v[pl.ds(g * _L, _L)]
                for l in range(_L):
                    wv = jnp.full((_L,), wg[l], jnp.float32)
                    r = g * _L + l
                    for j in range(_D // _L):
                        sl = pl.ds(j * _L, _L)
                        rows_v[r, sl] = rows_v[r, sl] * wv
                return carry2
            lax.fori_loop(0, _B // _L, scale, 0)
            pltpu.sync_copy(rows_v, acc_sh.at[di_v], add=True)
            return carry
        lax.fori_loop(0, n_chunks, chunk, 0)
        plsc.subcore_barrier()
        pltpu.sync_copy(acc_sh.at[pl.ds(s * rows_per_sub, rows_per_sub)],
                        out_hbm.at[c, pl.ds(s * rows_per_sub, rows_per_sub)])

    return k


_scatter_e = _make_scatter(5120)
_scatter_v = _make_scatter(10240)


def kernel(v, e, W_v2e, b_v2e, W_e2v, b_e2v, n_weight, e_weight,
           n_reg_weight, e_reg_weight, e_reg_sum, n_reg_sum,
           vidx, eidx, ve_lists):
    ve0 = ve_lists[:, 0]
    ve1 = ve_lists[:, 1]
    nrw = n_reg_weight[:, 0]
    erw = e_reg_weight[:, 0]

    x = _dense_in(v, W_v2e, b_v2e.reshape(1, _D), n_weight, 2000)
    s = _scatter_e(x, ve0, nrw, eidx)
    e1, y = _dense_mid(e, s[0, :_NE], s[1, :_NE], e_reg_sum,
                       W_e2v, b_e2v.reshape(1, _D), e_weight, 1000)
    t = _scatter_v(y, ve1, erw, vidx)
    v2 = _combine(v, n_weight, t[0, :_NV], t[1, :_NV], n_reg_sum, 2000)
    return (v2, e1)


# speedup vs baseline: 8.4163x; 2.2877x over previous
import functools

import jax
import jax.numpy as jnp
from jax import lax
from jax.experimental import pallas as pl
from jax.experimental.pallas import tpu as pltpu
from jax.experimental.pallas import tpu_sc as plsc

_NV, _NE, _E, _D = 10000, 5000, 320000, 128
_NC, _NS, _L = 2, 16, 16
_NW = _NC * _NS
_B = 80


def _dense_in_body(v_ref, w_ref, b_ref, nw_ref, x_ref):
    x = lax.dot_general(v_ref[...], w_ref[...],
                        (((1,), (1,)), ((), ())),
                        preferred_element_type=jnp.float32)
    x_ref[...] = jnp.maximum(x + b_ref[...], 0.0) * nw_ref[...]


def _dense_in(v, W, b2, nw, blk):
    n = v.shape[0]
    grid = n // blk
    return pl.pallas_call(
        _dense_in_body,
        grid=(grid,),
        in_specs=[
            pl.BlockSpec((blk, _D), lambda i: (i, 0)),
            pl.BlockSpec((_D, _D), lambda i: (0, 0)),
            pl.BlockSpec((1, _D), lambda i: (0, 0)),
            pl.BlockSpec((blk, 1), lambda i: (i, 0)),
        ],
        out_specs=pl.BlockSpec((blk, _D), lambda i: (i, 0)),
        out_shape=jax.ShapeDtypeStruct((n, _D), jnp.float32),
    )(v, W, b2, nw)


def _dense_mid_body(e_ref, s0_ref, s1_ref, ers_ref, w_ref, b_ref, ew_ref,
                    e1_ref, y_ref):
    e1 = (e_ref[...] + s0_ref[...] + s1_ref[...]) / ers_ref[...]
    e1_ref[...] = e1
    y = lax.dot_general(e1, w_ref[...], (((1,), (1,)), ((), ())),
                        preferred_element_type=jnp.float32)
    y_ref[...] = jnp.maximum(y + b_ref[...], 0.0) * ew_ref[...]


def _dense_mid(e, s0, s1, ers, W, b2, ew, blk):
    n = e.shape[0]
    grid = n // blk
    row = lambda i: (i, 0)
    fixed = lambda i: (0, 0)
    return pl.pallas_call(
        _dense_mid_body,
        grid=(grid,),
        in_specs=[
            pl.BlockSpec((blk, _D), row),
            pl.BlockSpec((blk, _D), row),
            pl.BlockSpec((blk, _D), row),
            pl.BlockSpec((blk, 1), row),
            pl.BlockSpec((_D, _D), fixed),
            pl.BlockSpec((1, _D), fixed),
            pl.BlockSpec((blk, 1), row),
        ],
        out_specs=[pl.BlockSpec((blk, _D), row), pl.BlockSpec((blk, _D), row)],
        out_shape=[jax.ShapeDtypeStruct((n, _D), jnp.float32),
                   jax.ShapeDtypeStruct((n, _D), jnp.float32)],
    )(e, s0, s1, ers, W, b2, ew)


def _combine_body(v_ref, nw_ref, t0_ref, t1_ref, nrs_ref, out_ref):
    out_ref[...] = (v_ref[...] * nw_ref[...] + t0_ref[...] + t1_ref[...]) \
        / nrs_ref[...]


def _combine(v, nw, t0, t1, nrs, blk):
    n = v.shape[0]
    grid = n // blk
    row = lambda i: (i, 0)
    return pl.pallas_call(
        _combine_body,
        grid=(grid,),
        in_specs=[
            pl.BlockSpec((blk, _D), row),
            pl.BlockSpec((blk, 1), row),
            pl.BlockSpec((blk, _D), row),
            pl.BlockSpec((blk, _D), row),
            pl.BlockSpec((blk, 1), row),
        ],
        out_specs=pl.BlockSpec((blk, _D), row),
        out_shape=jax.ShapeDtypeStruct((n, _D), jnp.float32),
    )(v, nw, t0, t1, nrs)


def _make_scatter(n_dst):
    per_w = _E // _NW
    n_chunks = per_w // _B
    rps = (n_dst // 8 // _NS) * 8
    tail = n_dst - _NS * rps
    mesh = plsc.VectorSubcoreMesh(core_axis_name="c", subcore_axis_name="s")

    @functools.partial(
        pl.kernel, mesh=mesh,
        out_type=jax.ShapeDtypeStruct((_NC, n_dst, _D), jnp.float32),
        scratch_types=[
            pltpu.VMEM((per_w * 2,), jnp.int32),
            pltpu.VMEM((per_w,), jnp.float32),
            pltpu.VMEM((2, _B, _D), jnp.float32),
            pltpu.VMEM((_B,), jnp.int32),
            pltpu.VMEM((_B,), jnp.int32),
            pltpu.VMEM((_B,), jnp.int32),
            pltpu.VMEM_SHARED((n_dst, _D), jnp.float32),
            pltpu.SemaphoreType.DMA,
            pltpu.SemaphoreType.DMA,
        ],
    )
    def k(x_hbm, idx_hbm, w_hbm, out_hbm, idx_v, w_v, rows_v,
          si0_v, si1_v, di_v, acc_sh, sem0, sem1):
        c = lax.axis_index("c")
        s = lax.axis_index("s")
        wid = s * _NC + c
        sems = (sem0, sem1)
        sis = (si0_v, si1_v)
        zero16 = jnp.zeros((_L,), jnp.float32)

        pltpu.sync_copy(idx_hbm.at[pl.ds(wid * per_w * 2, per_w * 2)], idx_v)
        pltpu.sync_copy(w_hbm.at[pl.ds(wid * per_w, per_w)], w_v)

        def zero_rows(r, carry):
            for j in range(_D // _L):
                rows_v[0, r, pl.ds(j * _L, _L)] = zero16
            return carry
        lax.fori_loop(0, _B, zero_rows, 0)

        def zero_range(base, length):
            for t in range(0, length, _B):
                nrows = min(_B, length - t)
                pltpu.sync_copy(rows_v.at[0, pl.ds(0, nrows)],
                                acc_sh.at[pl.ds(base + t, nrows)])
        zero_range(s * rps, rps)

        @pl.when(s == 0)
        def _():
            zero_range(_NS * rps, tail)
        plsc.subcore_barrier()

        def stage(dst_ref, off):
            for g in range(_B // _L):
                dst_ref[pl.ds(g * _L, _L)] = idx_v[pl.ds(off + g * _L, _L)]

        def gather(buf, si_ref, sem):
            return pltpu.make_async_copy(
                x_hbm.at[si_ref], rows_v.at[buf], sem)

        def process(t, buf):
            def scale(g, carry2):
                wg = w_v[pl.ds(t * _B + g * _L, _L)]
                for l in range(_L):
                    wv = jnp.full((_L,), wg[l], jnp.float32)
                    r = g * _L + l
                    for j in range(_D // _L):
                        sl = pl.ds(j * _L, _L)
                        rows_v[buf, r, sl] = rows_v[buf, r, sl] * wv
                return carry2
            lax.fori_loop(0, _B // _L, scale, 0)
            stage(di_v, t * 2 * _B + _B)
            pltpu.sync_copy(rows_v.at[buf], acc_sh.at[di_v], add=True)

        stage(sis[0], 0)
        gather(0, sis[0], sems[0]).start()

        def pair(p, carry):
            for par in range(2):
                t = p * 2 + par
                stage(sis[1 - par], (t + 1) * 2 * _B)
                gather(1 - par, sis[1 - par], sems[1 - par]).start()
                gather(par, sis[par], sems[par]).wait()
                process(t, par)
            return carry
        lax.fori_loop(0, (n_chunks - 1) // 2, pair, 0)
        t_last = n_chunks - 1
        gather(t_last % 2, sis[t_last % 2], sems[t_last % 2]).wait()
        process(t_last, t_last % 2)

        plsc.subcore_barrier()
        pltpu.sync_copy(acc_sh.at[pl.ds(s * rps, rps)],
                        out_hbm.at[c, pl.ds(s * rps, rps)])

        @pl.when(s == 0)
        def _():
            pltpu.sync_copy(acc_sh.at[pl.ds(_NS * rps, tail)],
                            out_hbm.at[c, pl.ds(_NS * rps, tail)])

    return k


_scatter_e = _make_scatter(_NE)
_scatter_v = _make_scatter(_NV)


def kernel(v, e, W_v2e, b_v2e, W_e2v, b_e2v, n_weight, e_weight,
           n_reg_weight, e_reg_weight, e_reg_sum, n_reg_sum,
           vidx, eidx, ve_lists):
    def pack(src, dst):
        return jnp.stack([src.reshape(-1, _B), dst.reshape(-1, _B)],
                         axis=1).reshape(-1)

    idx_e = pack(ve_lists[:, 0], eidx)
    idx_v = pack(ve_lists[:, 1], vidx)
    w_e = n_reg_weight[:, 0]
    w_v2 = e_reg_weight[:, 0]

    x = _dense_in(v, W_v2e, b_v2e.reshape(1, _D), n_weight, 2000)
    s = _scatter_e(x, idx_e, w_e)
    e1, y = _dense_mid(e, s[0], s[1], e_reg_sum,
                       W_e2v, b_e2v.reshape(1, _D), e_weight, 1000)
    t = _scatter_v(y, idx_v, w_v2)
    v2 = _combine(v, n_weight, t[0], t[1], n_reg_sum, 2000)
    return (v2, e1)


# speedup vs baseline: 9.4271x; 1.1201x over previous
import functools

import jax
import jax.numpy as jnp
from jax import lax
from jax.experimental import pallas as pl
from jax.experimental.pallas import tpu as pltpu
from jax.experimental.pallas import tpu_sc as plsc

_NV, _NE, _E, _D = 10000, 5000, 320000, 128
_NC, _NS, _L = 2, 16, 16
_NW = _NC * _NS
_B = 80


def _dense_in_body(v_ref, w_ref, b_ref, nw_ref, x_ref):
    x = lax.dot_general(v_ref[...], w_ref[...],
                        (((1,), (1,)), ((), ())),
                        preferred_element_type=jnp.float32)
    x_ref[...] = jnp.maximum(x + b_ref[...], 0.0) * nw_ref[...]


def _dense_in(v, W, b2, nw, blk):
    n = v.shape[0]
    grid = n // blk
    return pl.pallas_call(
        _dense_in_body,
        grid=(grid,),
        in_specs=[
            pl.BlockSpec((blk, _D), lambda i: (i, 0)),
            pl.BlockSpec((_D, _D), lambda i: (0, 0)),
            pl.BlockSpec((1, _D), lambda i: (0, 0)),
            pl.BlockSpec((blk, 1), lambda i: (i, 0)),
        ],
        out_specs=pl.BlockSpec((blk, _D), lambda i: (i, 0)),
        out_shape=jax.ShapeDtypeStruct((n, _D), jnp.float32),
    )(v, W, b2, nw)


def _dense_mid_body(e_ref, s0_ref, s1_ref, ers_ref, w_ref, b_ref, ew_ref,
                    e1_ref, y_ref):
    e1 = (e_ref[...] + s0_ref[...] + s1_ref[...]) / ers_ref[...]
    e1_ref[...] = e1
    y = lax.dot_general(e1, w_ref[...], (((1,), (1,)), ((), ())),
                        preferred_element_type=jnp.float32)
    y_ref[...] = jnp.maximum(y + b_ref[...], 0.0) * ew_ref[...]


def _dense_mid(e, s0, s1, ers, W, b2, ew, blk):
    n = e.shape[0]
    grid = n // blk
    row = lambda i: (i, 0)
    fixed = lambda i: (0, 0)
    return pl.pallas_call(
        _dense_mid_body,
        grid=(grid,),
        in_specs=[
            pl.BlockSpec((blk, _D), row),
            pl.BlockSpec((blk, _D), row),
            pl.BlockSpec((blk, _D), row),
            pl.BlockSpec((blk, 1), row),
            pl.BlockSpec((_D, _D), fixed),
            pl.BlockSpec((1, _D), fixed),
            pl.BlockSpec((blk, 1), row),
        ],
        out_specs=[pl.BlockSpec((blk, _D), row), pl.BlockSpec((blk, _D), row)],
        out_shape=[jax.ShapeDtypeStruct((n, _D), jnp.float32),
                   jax.ShapeDtypeStruct((n, _D), jnp.float32)],
    )(e, s0, s1, ers, W, b2, ew)


def _combine_body(v_ref, nw_ref, t0_ref, t1_ref, nrs_ref, out_ref):
    out_ref[...] = (v_ref[...] * nw_ref[...] + t0_ref[...] + t1_ref[...]) \
        / nrs_ref[...]


def _combine(v, nw, t0, t1, nrs, blk):
    n = v.shape[0]
    grid = n // blk
    row = lambda i: (i, 0)
    return pl.pallas_call(
        _combine_body,
        grid=(grid,),
        in_specs=[
            pl.BlockSpec((blk, _D), row),
            pl.BlockSpec((blk, 1), row),
            pl.BlockSpec((blk, _D), row),
            pl.BlockSpec((blk, _D), row),
            pl.BlockSpec((blk, 1), row),
        ],
        out_specs=pl.BlockSpec((blk, _D), row),
        out_shape=jax.ShapeDtypeStruct((n, _D), jnp.float32),
    )(v, nw, t0, t1, nrs)


def _make_scatter(n_dst):
    per_w = _E // _NW
    n_chunks = per_w // _B
    rps = (n_dst // 8 // _NS) * 8
    tail = n_dst - _NS * rps
    mesh = plsc.VectorSubcoreMesh(core_axis_name="c", subcore_axis_name="s")

    @functools.partial(
        pl.kernel, mesh=mesh,
        out_type=jax.ShapeDtypeStruct((_NC, n_dst, _D), jnp.float32),
        scratch_types=[
            pltpu.VMEM((3, _B, _D), jnp.float32),
            pltpu.VMEM((2 * _B,), jnp.int32),
            pltpu.VMEM((2 * _B,), jnp.int32),
            pltpu.VMEM((2 * _B,), jnp.int32),
            pltpu.VMEM((_B,), jnp.float32),
            pltpu.VMEM((_B,), jnp.float32),
            pltpu.VMEM((_B,), jnp.float32),
            pltpu.VMEM((_B,), jnp.int32),
            pltpu.VMEM((_B,), jnp.int32),
            pltpu.VMEM((_B,), jnp.int32),
            pltpu.VMEM((_B,), jnp.int32),
            pltpu.VMEM((_B,), jnp.int32),
            pltpu.VMEM((_B,), jnp.int32),
            pltpu.VMEM_SHARED((n_dst, _D), jnp.float32),
            pltpu.SemaphoreType.DMA,
            pltpu.SemaphoreType.DMA,
            pltpu.SemaphoreType.DMA,
            pltpu.SemaphoreType.DMA,
            pltpu.SemaphoreType.DMA,
            pltpu.SemaphoreType.DMA,
            pltpu.SemaphoreType.DMA,
            pltpu.SemaphoreType.DMA,
            pltpu.SemaphoreType.DMA,
        ],
    )
    def k(x_hbm, idx_hbm, w_hbm, out_hbm, rows_v,
          rc0, rc1, rc2, wv0, wv1, wv2, si0, si1, si2, di0, di1, di2,
          acc_sh, rs0, rs1, rs2, gs0, gs1, gs2, ss0, ss1, ss2):
        c = lax.axis_index("c")
        s = lax.axis_index("s")
        wid = s * _NC + c
        rcs = (rc0, rc1, rc2)
        wvs = (wv0, wv1, wv2)
        sis = (si0, si1, si2)
        dis = (di0, di1, di2)
        rse = (rs0, rs1, rs2)
        gse = (gs0, gs1, gs2)
        sse = (ss0, ss1, ss2)
        zero16 = jnp.zeros((_L,), jnp.float32)

        def rec_descs(t, b):
            return (
                pltpu.make_async_copy(
                    idx_hbm.at[pl.ds(wid * per_w * 2 + t * 2 * _B, 2 * _B)],
                    rcs[b], rse[b]),
                pltpu.make_async_copy(
                    w_hbm.at[pl.ds(wid * per_w + t * _B, _B)],
                    wvs[b], rse[b]),
            )

        def start_rec(t, b):
            for d in rec_descs(t, b):
                d.start()

        def wait_rec(t, b):
            for d in rec_descs(t, b):
                d.wait()

        def stage(dst_ref, src_ref, off):
            for g in range(_B // _L):
                dst_ref[pl.ds(g * _L, _L)] = src_ref[pl.ds(off + g * _L, _L)]

        def start_gather(t, b):
            wait_rec(t, b)
            stage(sis[b], rcs[b], 0)
            pltpu.async_copy(x_hbm.at[sis[b]], rows_v.at[b], gse[b])

        def wait_gather(b):
            pltpu.make_async_copy(
                x_hbm.at[sis[b]], rows_v.at[b], gse[b]).wait()

        def start_scatter(b):
            stage(dis[b], rcs[b], _B)
            pltpu.async_copy(rows_v.at[b], acc_sh.at[dis[b]], sse[b],
                             add=True)

        def wait_scatter(b):
            pltpu.make_async_copy(
                rows_v.at[b], acc_sh.at[dis[b]], sse[b]).wait()

        def scale(b):
            def scale_g(g, carry2):
                wg = wvs[b][pl.ds(g * _L, _L)]
                for l in range(_L):
                    wv = jnp.full((_L,), wg[l], jnp.float32)
                    r = g * _L + l
                    for j in range(_D // _L):
                        sl = pl.ds(j * _L, _L)
                        rows_v[b, r, sl] = rows_v[b, r, sl] * wv
                return carry2
            lax.fori_loop(0, _B // _L, scale_g, 0)

        start_rec(0, 0)
        start_rec(1, 1)
        start_rec(2, 2)

        def zero_rows(r, carry):
            for j in range(_D // _L):
                rows_v[0, r, pl.ds(j * _L, _L)] = zero16
            return carry
        lax.fori_loop(0, _B, zero_rows, 0)

        def zero_range(base, length):
            for t in range(0, length, _B):
                nrows = min(_B, length - t)
                pltpu.sync_copy(rows_v.at[0, pl.ds(0, nrows)],
                                acc_sh.at[pl.ds(base + t, nrows)])
        zero_range(s * rps, rps)

        @pl.when(s == 0)
        def _():
            zero_range(_NS * rps, tail)
        plsc.subcore_barrier()

        start_gather(0, 0)
        start_gather(1, 1)

        wait_gather(0)
        scale(0)
        start_scatter(0)
        start_gather(2, 2)
        start_rec(3, 0)

        n_main = (n_chunks - 5) // 3

        def triple(p, carry):
            for o in range(3):
                t = 1 + p * 3 + o
                b = (1 + o) % 3
                bp = o % 3
                wait_gather(b)
                scale(b)
                start_scatter(b)
                wait_scatter(bp)
                start_gather(t + 2, bp)
                start_rec(t + 3, b)
            return carry
        lax.fori_loop(0, n_main, triple, 0)

        for t in range(1 + 3 * n_main, n_chunks):
            b = t % 3
            wait_gather(b)
            scale(b)
            start_scatter(b)
            if t + 2 < n_chunks:
                bp = (t + 2) % 3
                wait_scatter(bp)
                start_gather(t + 2, bp)
            if t + 3 < n_chunks:
                start_rec(t + 3, b)
        for b in range(3):
            wait_scatter(b)

        plsc.subcore_barrier()
        pltpu.sync_copy(acc_sh.at[pl.ds(s * rps, rps)],
                        out_hbm.at[c, pl.ds(s * rps, rps)])

        @pl.when(s == 0)
        def _():
            pltpu.sync_copy(acc_sh.at[pl.ds(_NS * rps, tail)],
                            out_hbm.at[c, pl.ds(_NS * rps, tail)])

    return k


_scatter_e = _make_scatter(_NE)
_scatter_v = _make_scatter(_NV)


def kernel(v, e, W_v2e, b_v2e, W_e2v, b_e2v, n_weight, e_weight,
           n_reg_weight, e_reg_weight, e_reg_sum, n_reg_sum,
           vidx, eidx, ve_lists):
    def pack(src, dst):
        return jnp.stack([src.reshape(-1, _B), dst.reshape(-1, _B)],
                         axis=1).reshape(-1)

    idx_e = pack(ve_lists[:, 0], eidx)
    idx_v = pack(ve_lists[:, 1], vidx)
    w_e = n_reg_weight[:, 0]
    w_v2 = e_reg_weight[:, 0]

    x = _dense_in(v, W_v2e, b_v2e.reshape(1, _D), n_weight, 2000)
    s = _scatter_e(x, idx_e, w_e)
    e1, y = _dense_mid(e, s[0], s[1], e_reg_sum,
                       W_e2v, b_e2v.reshape(1, _D), e_weight, 1000)
    t = _scatter_v(y, idx_v, w_v2)
    v2 = _combine(v, n_weight, t[0], t[1], n_reg_sum, 2000)
    return (v2, e1)


# speedup vs baseline: 10.2604x; 1.0884x over previous
import functools

import jax
import jax.numpy as jnp
from jax import lax
from jax.experimental import pallas as pl
from jax.experimental.pallas import tpu as pltpu
from jax.experimental.pallas import tpu_sc as plsc

_NV, _NE, _E, _D = 10000, 5000, 320000, 128
_NC, _NS, _L = 2, 16, 16
_NW = _NC * _NS
_B = 80


def _dense_in_body(v_ref, w_ref, b_ref, nw_ref, x_ref):
    x = lax.dot_general(v_ref[...], w_ref[...],
                        (((1,), (1,)), ((), ())),
                        preferred_element_type=jnp.float32)
    x_ref[...] = jnp.maximum(x + b_ref[...], 0.0) * nw_ref[...]


def _dense_in(v, W, b2, nw, blk):
    n = v.shape[0]
    grid = n // blk
    return pl.pallas_call(
        _dense_in_body,
        grid=(grid,),
        in_specs=[
            pl.BlockSpec((blk, _D), lambda i: (i, 0)),
            pl.BlockSpec((_D, _D), lambda i: (0, 0)),
            pl.BlockSpec((1, _D), lambda i: (0, 0)),
            pl.BlockSpec((blk, 1), lambda i: (i, 0)),
        ],
        out_specs=pl.BlockSpec((blk, _D), lambda i: (i, 0)),
        out_shape=jax.ShapeDtypeStruct((n, _D), jnp.float32),
    )(v, W, b2, nw)


def _dense_mid_body(e_ref, s0_ref, s1_ref, ers_ref, w_ref, b_ref, ew_ref,
                    e1_ref, y_ref):
    e1 = (e_ref[...] + s0_ref[...] + s1_ref[...]) / ers_ref[...]
    e1_ref[...] = e1
    y = lax.dot_general(e1, w_ref[...], (((1,), (1,)), ((), ())),
                        preferred_element_type=jnp.float32)
    y_ref[...] = jnp.maximum(y + b_ref[...], 0.0) * ew_ref[...]


def _dense_mid(e, s0, s1, ers, W, b2, ew, blk):
    n = e.shape[0]
    grid = n // blk
    row = lambda i: (i, 0)
    fixed = lambda i: (0, 0)
    return pl.pallas_call(
        _dense_mid_body,
        grid=(grid,),
        in_specs=[
            pl.BlockSpec((blk, _D), row),
            pl.BlockSpec((blk, _D), row),
            pl.BlockSpec((blk, _D), row),
            pl.BlockSpec((blk, 1), row),
            pl.BlockSpec((_D, _D), fixed),
            pl.BlockSpec((1, _D), fixed),
            pl.BlockSpec((blk, 1), row),
        ],
        out_specs=[pl.BlockSpec((blk, _D), row), pl.BlockSpec((blk, _D), row)],
        out_shape=[jax.ShapeDtypeStruct((n, _D), jnp.float32),
                   jax.ShapeDtypeStruct((n, _D), jnp.float32)],
    )(e, s0, s1, ers, W, b2, ew)


def _combine_body(v_ref, nw_ref, t0_ref, t1_ref, nrs_ref, out_ref):
    out_ref[...] = (v_ref[...] * nw_ref[...] + t0_ref[...] + t1_ref[...]) \
        / nrs_ref[...]


def _combine(v, nw, t0, t1, nrs, blk):
    n = v.shape[0]
    grid = n // blk
    row = lambda i: (i, 0)
    return pl.pallas_call(
        _combine_body,
        grid=(grid,),
        in_specs=[
            pl.BlockSpec((blk, _D), row),
            pl.BlockSpec((blk, 1), row),
            pl.BlockSpec((blk, _D), row),
            pl.BlockSpec((blk, _D), row),
            pl.BlockSpec((blk, 1), row),
        ],
        out_specs=pl.BlockSpec((blk, _D), row),
        out_shape=jax.ShapeDtypeStruct((n, _D), jnp.float32),
    )(v, nw, t0, t1, nrs)


def _make_scatter(n_dst):
    per_w = _E // _NW
    n_chunks = per_w // _B
    rps = (n_dst // 8 // _NS) * 8
    tail = n_dst - _NS * rps
    mesh = plsc.VectorSubcoreMesh(core_axis_name="c", subcore_axis_name="s")

    @functools.partial(
        pl.kernel, mesh=mesh,
        out_type=jax.ShapeDtypeStruct((_NC, n_dst, _D), jnp.float32),
        scratch_types=[
            pltpu.VMEM((3, _B, _D), jnp.float32),
            pltpu.VMEM((_B,), jnp.float32),
            pltpu.VMEM((_B,), jnp.float32),
            pltpu.VMEM((_B,), jnp.float32),
            pltpu.VMEM((_B,), jnp.int32),
            pltpu.VMEM((_B,), jnp.int32),
            pltpu.VMEM((_B,), jnp.int32),
            pltpu.VMEM((_B,), jnp.int32),
            pltpu.VMEM((_B,), jnp.int32),
            pltpu.VMEM((_B,), jnp.int32),
            pltpu.VMEM_SHARED((n_dst, _D), jnp.float32),
            pltpu.SemaphoreType.DMA,
            pltpu.SemaphoreType.DMA,
            pltpu.SemaphoreType.DMA,
            pltpu.SemaphoreType.DMA,
            pltpu.SemaphoreType.DMA,
            pltpu.SemaphoreType.DMA,
            pltpu.SemaphoreType.DMA,
            pltpu.SemaphoreType.DMA,
            pltpu.SemaphoreType.DMA,
        ],
    )
    def k(x_hbm, si_hbm, di_hbm, w_hbm, out_hbm, rows_v,
          wv0, wv1, wv2, si0, si1, si2, di0, di1, di2,
          acc_sh, rs0, rs1, rs2, gs0, gs1, gs2, ss0, ss1, ss2):
        c = lax.axis_index("c")
        s = lax.axis_index("s")
        wid = s * _NC + c
        wvs = (wv0, wv1, wv2)
        sis = (si0, si1, si2)
        dis = (di0, di1, di2)
        rse = (rs0, rs1, rs2)
        gse = (gs0, gs1, gs2)
        sse = (ss0, ss1, ss2)
        zero16 = jnp.zeros((_L,), jnp.float32)

        def rec_descs(t, b):
            base = wid * per_w + t * _B
            return (
                pltpu.make_async_copy(
                    si_hbm.at[pl.ds(base, _B)], sis[b], rse[b]),
                pltpu.make_async_copy(
                    di_hbm.at[pl.ds(base, _B)], dis[b], rse[b]),
                pltpu.make_async_copy(
                    w_hbm.at[pl.ds(base, _B)], wvs[b], rse[b]),
            )

        def start_rec(t, b):
            for d in rec_descs(t, b):
                d.start()

        def wait_rec(t, b):
            for d in rec_descs(t, b):
                d.wait()

        def start_gather(t, b):
            wait_rec(t, b)
            pltpu.async_copy(x_hbm.at[sis[b]], rows_v.at[b], gse[b])

        def wait_gather(b):
            pltpu.make_async_copy(
                x_hbm.at[sis[b]], rows_v.at[b], gse[b]).wait()

        def start_scatter(b):
            pltpu.async_copy(rows_v.at[b], acc_sh.at[dis[b]], sse[b],
                             add=True)

        def wait_scatter(b):
            pltpu.make_async_copy(
                rows_v.at[b], acc_sh.at[dis[b]], sse[b]).wait()

        def scale(b):
            def scale_g(g, carry2):
                wg = wvs[b][pl.ds(g * _L, _L)]
                for l in range(_L):
                    wv = jnp.full((_L,), wg[l], jnp.float32)
                    r = g * _L + l
                    for j in range(_D // _L):
                        sl = pl.ds(j * _L, _L)
                        rows_v[b, r, sl] = rows_v[b, r, sl] * wv
                return carry2
            lax.fori_loop(0, _B // _L, scale_g, 0)

        start_rec(0, 0)
        start_rec(1, 1)
        start_rec(2, 2)

        def zero_rows(r, carry):
            for j in range(_D // _L):
                rows_v[0, r, pl.ds(j * _L, _L)] = zero16
            return carry
        lax.fori_loop(0, _B, zero_rows, 0)

        def zero_range(base, length):
            for t in range(0, length, _B):
                nrows = min(_B, length - t)
                pltpu.sync_copy(rows_v.at[0, pl.ds(0, nrows)],
                                acc_sh.at[pl.ds(base + t, nrows)])
        zero_range(s * rps, rps)

        @pl.when(s == 0)
        def _():
            zero_range(_NS * rps, tail)
        plsc.subcore_barrier()

        start_gather(0, 0)
        start_gather(1, 1)

        wait_gather(0)
        scale(0)
        start_scatter(0)
        start_gather(2, 2)
        start_rec(3, 0)

        n_main = (n_chunks - 5) // 3

        def triple(p, carry):
            for o in range(3):
                t = 1 + p * 3 + o
                b = (1 + o) % 3
                bp = o % 3
                wait_gather(b)
                scale(b)
                start_scatter(b)
                wait_scatter(bp)
                start_gather(t + 2, bp)
                start_rec(t + 3, b)
            return carry
        lax.fori_loop(0, n_main, triple, 0)

        for t in range(1 + 3 * n_main, n_chunks):
            b = t % 3
            wait_gather(b)
            scale(b)
            start_scatter(b)
            if t + 2 < n_chunks:
                bp = (t + 2) % 3
                wait_scatter(bp)
                start_gather(t + 2, bp)
            if t + 3 < n_chunks:
                start_rec(t + 3, b)
        for b in range(3):
            wait_scatter(b)

        plsc.subcore_barrier()
        pltpu.sync_copy(acc_sh.at[pl.ds(s * rps, rps)],
                        out_hbm.at[c, pl.ds(s * rps, rps)])

        @pl.when(s == 0)
        def _():
            pltpu.sync_copy(acc_sh.at[pl.ds(_NS * rps, tail)],
                            out_hbm.at[c, pl.ds(_NS * rps, tail)])

    return k


_scatter_e = _make_scatter(_NE)
_scatter_v = _make_scatter(_NV)


def kernel(v, e, W_v2e, b_v2e, W_e2v, b_e2v, n_weight, e_weight,
           n_reg_weight, e_reg_weight, e_reg_sum, n_reg_sum,
           vidx, eidx, ve_lists):
    ve0 = jnp.asarray(ve_lists[:, 0])
    ve1 = jnp.asarray(ve_lists[:, 1])
    w_e = n_reg_weight[:, 0]
    w_v2 = e_reg_weight[:, 0]

    x = _dense_in(v, W_v2e, b_v2e.reshape(1, _D), n_weight, 2000)
    s = _scatter_e(x, ve0, eidx, w_e)
    e1, y = _dense_mid(e, s[0], s[1], e_reg_sum,
                       W_e2v, b_e2v.reshape(1, _D), e_weight, 1000)
    t = _scatter_v(y, ve1, vidx, w_v2)
    v2 = _combine(v, n_weight, t[0], t[1], n_reg_sum, 2000)
    return (v2, e1)
